# Initial kernel scaffold; baseline (speedup 1.0000x reference)
import functools

import jax
import jax.numpy as jnp
from jax import lax
from jax.experimental import pallas as pl
from jax.experimental.pallas import tpu as pltpu
from jax.experimental.pallas import tpu_sc as plsc

N = 10000
E = 320000
D = 128

NC = 2
NS = 16
NW = NC * NS
EPW = E // NW
CH = 128
NFULL = EPW // CH
TAIL = EPW - NFULL * CH
RPT = N // NS

_mesh = plsc.VectorSubcoreMesh(core_axis_name="c", subcore_axis_name="s")


@functools.partial(
    pl.kernel,
    out_type=jax.ShapeDtypeStruct((NC, N), jnp.float32),
    mesh=_mesh,
    scratch_types=[
        pltpu.VMEM((CH,), jnp.int32),
        pltpu.VMEM((CH,), jnp.float32),
        pltpu.VMEM((TAIL,), jnp.int32),
        pltpu.VMEM((TAIL,), jnp.float32),
        pltpu.VMEM_SHARED((N,), jnp.float32),
    ],
)
def _deg_kernel(dst_hbm, zeros1_hbm, out_hbm, idx_v, ones_v, idxt_v, onest_v,
                deg_sh):
    cid = lax.axis_index("c")
    sid = lax.axis_index("s")
    wid = sid * NC + cid

    @pl.when(sid == 0)
    def _():
        pltpu.sync_copy(zeros1_hbm, deg_sh)

    for i in range(CH // 16):
        ones_v[pl.ds(i * 16, 16)] = jnp.ones((16,), jnp.float32)
    onest_v[...] = jnp.ones((TAIL,), jnp.float32)
    plsc.subcore_barrier()

    def chunk(j, _):
        base = wid * EPW + j * CH
        pltpu.sync_copy(dst_hbm.at[pl.ds(base, CH)], idx_v)
        pltpu.sync_copy(ones_v, deg_sh.at[idx_v], add=True)
        return 0

    lax.fori_loop(0, NFULL, chunk, 0)
    tbase = wid * EPW + NFULL * CH
    pltpu.sync_copy(dst_hbm.at[pl.ds(tbase, TAIL)], idxt_v)
    pltpu.sync_copy(onest_v, deg_sh.at[idxt_v], add=True)

    plsc.subcore_barrier()

    @pl.when(sid == 0)
    def _():
        pltpu.sync_copy(deg_sh, out_hbm.at[cid])


@functools.partial(
    pl.kernel,
    out_type=jax.ShapeDtypeStruct((NC, N, D), jnp.float32),
    mesh=_mesh,
    scratch_types=[
        pltpu.VMEM((CH,), jnp.int32),
        pltpu.VMEM((CH,), jnp.int32),
        pltpu.VMEM((CH, D), jnp.float32),
        pltpu.VMEM((TAIL,), jnp.int32),
        pltpu.VMEM((TAIL,), jnp.int32),
        pltpu.VMEM((TAIL, D), jnp.float32),
        pltpu.VMEM_SHARED((N, D), jnp.float32),
        pltpu.SemaphoreType.DMA,
    ],
)
def _agg_kernel(src_hbm, dst_hbm, y_hbm, zeros2_hbm, out_hbm,
                sidx_v, didx_v, rows_v, sidxt_v, didxt_v, rowst_v,
                acc_sh, sem):
    cid = lax.axis_index("c")
    sid = lax.axis_index("s")
    wid = sid * NC + cid
    r0 = sid * RPT

    @pl.when(cid == 0)
    def _():
        pltpu.sync_copy(y_hbm.at[pl.ds(r0, RPT)], acc_sh.at[pl.ds(r0, RPT)])

    @pl.when(cid == 1)
    def _():
        pltpu.sync_copy(zeros2_hbm.at[pl.ds(r0, RPT)],
                        acc_sh.at[pl.ds(r0, RPT)])

    plsc.subcore_barrier()

    def chunk(j, _):
        base = wid * EPW + j * CH
        pltpu.sync_copy(src_hbm.at[pl.ds(base, CH)], sidx_v)
        pltpu.sync_copy(dst_hbm.at[pl.ds(base, CH)], didx_v)
        pltpu.async_copy(y_hbm.at[sidx_v], rows_v, sem).wait()
        pltpu.sync_copy(rows_v, acc_sh.at[didx_v], add=True)
        return 0

    lax.fori_loop(0, NFULL, chunk, 0)
    tbase = wid * EPW + NFULL * CH
    pltpu.sync_copy(src_hbm.at[pl.ds(tbase, TAIL)], sidxt_v)
    pltpu.sync_copy(dst_hbm.at[pl.ds(tbase, TAIL)], didxt_v)
    pltpu.async_copy(y_hbm.at[sidxt_v], rowst_v, sem).wait()
    pltpu.sync_copy(rowst_v, acc_sh.at[didxt_v], add=True)

    plsc.subcore_barrier()
    pltpu.sync_copy(acc_sh.at[pl.ds(r0, RPT)], out_hbm.at[cid, pl.ds(r0, RPT)])


BR = 1000


def _y_body(deg_ref, x_ref, w_ref, y_ref):
    d = deg_ref[0, :, 0] + deg_ref[1, :, 0] + 1.0
    dinv = lax.rsqrt(d)
    y_ref[...] = jnp.dot(x_ref[...] * dinv[:, None], w_ref[...],
                         preferred_element_type=jnp.float32)


_y_call = pl.pallas_call(
    _y_body,
    grid=(N // BR,),
    in_specs=[
        pl.BlockSpec((NC, BR, 1), lambda i: (0, i, 0)),
        pl.BlockSpec((BR, D), lambda i: (i, 0)),
        pl.BlockSpec((D, D), lambda i: (0, 0)),
    ],
    out_specs=pl.BlockSpec((BR, D), lambda i: (i, 0)),
    out_shape=jax.ShapeDtypeStruct((N, D), jnp.float32),
)


def _out_body(deg_ref, acc_ref, b_ref, o_ref):
    d = deg_ref[0, :, 0] + deg_ref[1, :, 0] + 1.0
    dinv = lax.rsqrt(d)
    o_ref[...] = (acc_ref[0] + acc_ref[1]) * dinv[:, None] + b_ref[...]


_out_call = pl.pallas_call(
    _out_body,
    grid=(N // BR,),
    in_specs=[
        pl.BlockSpec((NC, BR, 1), lambda i: (0, i, 0)),
        pl.BlockSpec((NC, BR, D), lambda i: (0, i, 0)),
        pl.BlockSpec((1, D), lambda i: (0, 0)),
    ],
    out_specs=pl.BlockSpec((BR, D), lambda i: (i, 0)),
    out_shape=jax.ShapeDtypeStruct((N, D), jnp.float32),
)


def kernel(x, edge_index, W, b):
    ei = edge_index.astype(jnp.int32)
    src = ei[0]
    dst = ei[1]
    zeros1 = jnp.zeros((N,), jnp.float32)
    zeros2 = jnp.zeros((N, D), jnp.float32)
    degp = _deg_kernel(dst, zeros1)
    degp3 = degp.reshape(NC, N, 1)
    y = _y_call(degp3, x, W)
    accp = _agg_kernel(src, dst, y, zeros2)
    return _out_call(degp3, accp, b.reshape(1, D))


# speedup vs baseline: 22.1126x; 22.1126x over previous
import functools

import jax
import jax.numpy as jnp
from jax import lax
from jax.experimental import pallas as pl
from jax.experimental.pallas import tpu as pltpu
from jax.experimental.pallas import tpu_sc as plsc

N = 10000
E = 320000
D = 128

NC = 2
NS = 16
NW = NC * NS
EPW = E // NW
CH = 128
NFULL = EPW // CH
TAIL = EPW - NFULL * CH
RPT = 632
RPT_LAST = N - (NS - 1) * RPT

_mesh = plsc.VectorSubcoreMesh(core_axis_name="c", subcore_axis_name="s")


@functools.partial(
    pl.kernel,
    out_type=jax.ShapeDtypeStruct((NC, N), jnp.float32),
    mesh=_mesh,
    scratch_types=[
        pltpu.VMEM((CH,), jnp.int32),
        pltpu.VMEM((CH,), jnp.float32),
        pltpu.VMEM((TAIL,), jnp.int32),
        pltpu.VMEM((TAIL,), jnp.float32),
        pltpu.VMEM_SHARED((N,), jnp.float32),
    ],
)
def _deg_kernel(dst_hbm, zeros1_hbm, out_hbm, idx_v, ones_v, idxt_v, onest_v,
                deg_sh):
    cid = lax.axis_index("c")
    sid = lax.axis_index("s")
    wid = sid * NC + cid

    @pl.when(sid == 0)
    def _():
        pltpu.sync_copy(zeros1_hbm, deg_sh)

    for i in range(CH // 16):
        ones_v[pl.ds(i * 16, 16)] = jnp.ones((16,), jnp.float32)
    onest_v[...] = jnp.ones((TAIL,), jnp.float32)
    plsc.subcore_barrier()

    def chunk(j, _):
        base = wid * EPW + j * CH
        pltpu.sync_copy(dst_hbm.at[pl.ds(base, CH)], idx_v)
        pltpu.sync_copy(ones_v, deg_sh.at[idx_v], add=True)
        return 0

    lax.fori_loop(0, NFULL, chunk, 0)
    tbase = wid * EPW + NFULL * CH
    pltpu.sync_copy(dst_hbm.at[pl.ds(tbase, TAIL)], idxt_v)
    pltpu.sync_copy(onest_v, deg_sh.at[idxt_v], add=True)

    plsc.subcore_barrier()

    @pl.when(sid == 0)
    def _():
        pltpu.sync_copy(deg_sh, out_hbm.at[cid])


@functools.partial(
    pl.kernel,
    out_type=jax.ShapeDtypeStruct((NC, N, D), jnp.float32),
    mesh=_mesh,
    scratch_types=[
        pltpu.VMEM((CH,), jnp.int32),
        pltpu.VMEM((CH,), jnp.int32),
        pltpu.VMEM((CH, D), jnp.float32),
        pltpu.VMEM((TAIL,), jnp.int32),
        pltpu.VMEM((TAIL,), jnp.int32),
        pltpu.VMEM((TAIL, D), jnp.float32),
        pltpu.VMEM_SHARED((N, D), jnp.float32),
        pltpu.SemaphoreType.DMA,
    ],
)
def _agg_kernel(src_hbm, dst_hbm, y_hbm, zeros2_hbm, out_hbm,
                sidx_v, didx_v, rows_v, sidxt_v, didxt_v, rowst_v,
                acc_sh, sem):
    cid = lax.axis_index("c")
    sid = lax.axis_index("s")
    wid = sid * NC + cid
    r0 = sid * RPT

    init_hbm = [y_hbm, zeros2_hbm]
    for c in range(NC):
        @pl.when((cid == c) & (sid < NS - 1))
        def _(c=c):
            pltpu.sync_copy(init_hbm[c].at[pl.ds(r0, RPT)],
                            acc_sh.at[pl.ds(r0, RPT)])

        @pl.when((cid == c) & (sid == NS - 1))
        def _(c=c):
            pltpu.sync_copy(init_hbm[c].at[pl.ds(r0, RPT_LAST)],
                            acc_sh.at[pl.ds(r0, RPT_LAST)])

    plsc.subcore_barrier()

    def chunk(j, _):
        base = wid * EPW + j * CH
        pltpu.sync_copy(src_hbm.at[pl.ds(base, CH)], sidx_v)
        pltpu.sync_copy(dst_hbm.at[pl.ds(base, CH)], didx_v)
        pltpu.async_copy(y_hbm.at[sidx_v], rows_v, sem).wait()
        pltpu.sync_copy(rows_v, acc_sh.at[didx_v], add=True)
        return 0

    lax.fori_loop(0, NFULL, chunk, 0)
    tbase = wid * EPW + NFULL * CH
    pltpu.sync_copy(src_hbm.at[pl.ds(tbase, TAIL)], sidxt_v)
    pltpu.sync_copy(dst_hbm.at[pl.ds(tbase, TAIL)], didxt_v)
    pltpu.async_copy(y_hbm.at[sidxt_v], rowst_v, sem).wait()
    pltpu.sync_copy(rowst_v, acc_sh.at[didxt_v], add=True)

    plsc.subcore_barrier()

    @pl.when(sid < NS - 1)
    def _():
        pltpu.sync_copy(acc_sh.at[pl.ds(r0, RPT)],
                        out_hbm.at[cid, pl.ds(r0, RPT)])

    @pl.when(sid == NS - 1)
    def _():
        pltpu.sync_copy(acc_sh.at[pl.ds(r0, RPT_LAST)],
                        out_hbm.at[cid, pl.ds(r0, RPT_LAST)])


BR = 1000


def _y_body(deg_ref, x_ref, w_ref, y_ref):
    d = deg_ref[0, :, 0] + deg_ref[1, :, 0] + 1.0
    dinv = lax.rsqrt(d)
    y_ref[...] = jnp.dot(x_ref[...] * dinv[:, None], w_ref[...],
                         preferred_element_type=jnp.float32)


_y_call = pl.pallas_call(
    _y_body,
    grid=(N // BR,),
    in_specs=[
        pl.BlockSpec((NC, BR, 1), lambda i: (0, i, 0)),
        pl.BlockSpec((BR, D), lambda i: (i, 0)),
        pl.BlockSpec((D, D), lambda i: (0, 0)),
    ],
    out_specs=pl.BlockSpec((BR, D), lambda i: (i, 0)),
    out_shape=jax.ShapeDtypeStruct((N, D), jnp.float32),
)


def _out_body(deg_ref, acc_ref, b_ref, o_ref):
    d = deg_ref[0, :, 0] + deg_ref[1, :, 0] + 1.0
    dinv = lax.rsqrt(d)
    o_ref[...] = (acc_ref[0] + acc_ref[1]) * dinv[:, None] + b_ref[...]


_out_call = pl.pallas_call(
    _out_body,
    grid=(N // BR,),
    in_specs=[
        pl.BlockSpec((NC, BR, 1), lambda i: (0, i, 0)),
        pl.BlockSpec((NC, BR, D), lambda i: (0, i, 0)),
        pl.BlockSpec((1, D), lambda i: (0, 0)),
    ],
    out_specs=pl.BlockSpec((BR, D), lambda i: (i, 0)),
    out_shape=jax.ShapeDtypeStruct((N, D), jnp.float32),
)


def kernel(x, edge_index, W, b):
    ei = edge_index.astype(jnp.int32)
    src = ei[0]
    dst = ei[1]
    zeros1 = jnp.zeros((N,), jnp.float32)
    zeros2 = jnp.zeros((N, D), jnp.float32)
    degp = _deg_kernel(dst, zeros1)
    degp3 = degp.reshape(NC, N, 1)
    y = _y_call(degp3, x, W)
    accp = _agg_kernel(src, dst, y, zeros2)
    return _out_call(degp3, accp, b.reshape(1, D))


# speedup vs baseline: 39.2648x; 1.7757x over previous
import functools

import jax
import jax.numpy as jnp
from jax import lax
from jax.experimental import pallas as pl
from jax.experimental.pallas import tpu as pltpu
from jax.experimental.pallas import tpu_sc as plsc

N = 10000
E = 320000
D = 128

NC = 2
NS = 16
NW = NC * NS
EPW = E // NW
CH = 128
NCH = 81
EPT = NCH * CH
RING = 3
NSINK = 8

RPT = 632
RPT_LAST = N - (NS - 1) * RPT

_mesh = plsc.VectorSubcoreMesh(core_axis_name="c", subcore_axis_name="s")


@functools.partial(
    pl.kernel,
    out_type=jax.ShapeDtypeStruct((NC, N + NSINK), jnp.float32),
    mesh=_mesh,
    scratch_types=[
        pltpu.VMEM((NCH, 2, CH), jnp.int32),
        pltpu.VMEM((CH,), jnp.float32),
        pltpu.VMEM_SHARED((N + NSINK,), jnp.float32),
        pltpu.SemaphoreType.DMA,
    ],
)
def _deg_kernel(sd_hbm, zeros1_hbm, out_hbm, idx_v, ones_v, deg_sh, sem):
    cid = lax.axis_index("c")
    sid = lax.axis_index("s")
    wid = sid * NC + cid

    @pl.when(sid == 0)
    def _():
        pltpu.sync_copy(zeros1_hbm, deg_sh)

    pltpu.sync_copy(sd_hbm.at[wid], idx_v)
    for i in range(CH // 16):
        ones_v[pl.ds(i * 16, 16)] = jnp.ones((16,), jnp.float32)
    plsc.subcore_barrier()

    def fire(j, _):
        pltpu.async_copy(ones_v, deg_sh.at[idx_v.at[j, 1]], sem, add=True)
        return 0

    lax.fori_loop(0, NCH, fire, 0)

    def drain(j, _):
        pltpu.make_async_copy(ones_v, deg_sh.at[idx_v.at[j, 1]], sem).wait()
        return 0

    lax.fori_loop(0, NCH, drain, 0)

    plsc.subcore_barrier()

    @pl.when(sid == 0)
    def _():
        pltpu.sync_copy(deg_sh, out_hbm.at[cid])


@functools.partial(
    pl.kernel,
    out_type=jax.ShapeDtypeStruct((NC, N, D), jnp.float32),
    mesh=_mesh,
    scratch_types=[
        pltpu.VMEM((2, CH), jnp.int32),
        pltpu.VMEM((2, CH), jnp.int32),
        pltpu.VMEM((2, CH), jnp.int32),
        pltpu.VMEM((CH, D), jnp.float32),
        pltpu.VMEM((CH, D), jnp.float32),
        pltpu.VMEM((CH, D), jnp.float32),
        pltpu.VMEM_SHARED((N + NSINK, D), jnp.float32),
        pltpu.SemaphoreType.DMA,
        pltpu.SemaphoreType.DMA,
        pltpu.SemaphoreType.DMA,
        pltpu.SemaphoreType.DMA,
        pltpu.SemaphoreType.DMA,
        pltpu.SemaphoreType.DMA,
    ],
)
def _agg_kernel(sd_hbm, y_hbm, zeros2_hbm, out_hbm,
                idx0_v, idx1_v, idx2_v, rows0_v, rows1_v, rows2_v, acc_sh,
                semg0, semg1, semg2, semi0, semi1, semi2):
    cid = lax.axis_index("c")
    sid = lax.axis_index("s")
    wid = sid * NC + cid
    r0 = sid * RPT
    idx = [idx0_v, idx1_v, idx2_v]
    rows = [rows0_v, rows1_v, rows2_v]
    semg = [semg0, semg1, semg2]
    semi = [semi0, semi1, semi2]

    init_hbm = [y_hbm, zeros2_hbm]
    for c in range(NC):
        @pl.when((cid == c) & (sid < NS - 1))
        def _(c=c):
            pltpu.sync_copy(init_hbm[c].at[pl.ds(r0, RPT)],
                            acc_sh.at[pl.ds(r0, RPT)])

        @pl.when((cid == c) & (sid == NS - 1))
        def _(c=c):
            pltpu.sync_copy(init_hbm[c].at[pl.ds(r0, RPT_LAST)],
                            acc_sh.at[pl.ds(r0, RPT_LAST)])

    @pl.when(sid == 0)
    def _():
        pltpu.sync_copy(zeros2_hbm.at[pl.ds(0, NSINK)],
                        acc_sh.at[pl.ds(N, NSINK)])

    for b in range(RING):
        pltpu.async_copy(sd_hbm.at[wid, b], idx[b], semi[b])
    for b in range(2):
        pltpu.make_async_copy(sd_hbm.at[wid, b], idx[b], semi[b]).wait()
        pltpu.async_copy(y_hbm.at[idx[b].at[0]], rows[b], semg[b])

    plsc.subcore_barrier()

    def outer(g, _):
        for b in range(RING):
            j = g * RING + b
            nb = (b + 2) % RING

            @pl.when(j + 2 < NCH)
            def _(j=j, nb=nb):
                pltpu.make_async_copy(sd_hbm.at[wid, j + 2], idx[nb],
                                      semi[nb]).wait()
                pltpu.async_copy(y_hbm.at[idx[nb].at[0]], rows[nb], semg[nb])

            pltpu.make_async_copy(y_hbm.at[idx[b].at[0]], rows[b],
                                  semg[b]).wait()
            pltpu.sync_copy(rows[b], acc_sh.at[idx[b].at[1]], add=True)

            @pl.when(j + 3 < NCH)
            def _(j=j, b=b):
                pltpu.async_copy(sd_hbm.at[wid, j + 3], idx[b], semi[b])
        return 0

    lax.fori_loop(0, NCH // RING, outer, 0)

    plsc.subcore_barrier()

    @pl.when(sid < NS - 1)
    def _():
        pltpu.sync_copy(acc_sh.at[pl.ds(r0, RPT)],
                        out_hbm.at[cid, pl.ds(r0, RPT)])

    @pl.when(sid == NS - 1)
    def _():
        pltpu.sync_copy(acc_sh.at[pl.ds(r0, RPT_LAST)],
                        out_hbm.at[cid, pl.ds(r0, RPT_LAST)])


BR = 1000


def _xw_body(x_ref, w_ref, o_ref):
    o_ref[...] = jnp.dot(x_ref[...], w_ref[...],
                         preferred_element_type=jnp.float32)


_xw_call = pl.pallas_call(
    _xw_body,
    grid=(N // BR,),
    in_specs=[
        pl.BlockSpec((BR, D), lambda i: (i, 0)),
        pl.BlockSpec((D, D), lambda i: (0, 0)),
    ],
    out_specs=pl.BlockSpec((BR, D), lambda i: (i, 0)),
    out_shape=jax.ShapeDtypeStruct((N, D), jnp.float32),
)


def _y_body(deg_ref, xw_ref, y_ref):
    d = deg_ref[0, :, 0] + deg_ref[1, :, 0] + 1.0
    dinv = lax.rsqrt(d)
    y_ref[...] = xw_ref[...] * dinv[:, None]


_y_call = pl.pallas_call(
    _y_body,
    grid=(N // BR,),
    in_specs=[
        pl.BlockSpec((NC, BR, 1), lambda i: (0, i, 0)),
        pl.BlockSpec((BR, D), lambda i: (i, 0)),
    ],
    out_specs=pl.BlockSpec((BR, D), lambda i: (i, 0)),
    out_shape=jax.ShapeDtypeStruct((N, D), jnp.float32),
)


def _out_body(deg_ref, acc_ref, b_ref, o_ref):
    d = deg_ref[0, :, 0] + deg_ref[1, :, 0] + 1.0
    dinv = lax.rsqrt(d)
    o_ref[...] = (acc_ref[0] + acc_ref[1]) * dinv[:, None] + b_ref[...]


_out_call = pl.pallas_call(
    _out_body,
    grid=(N // BR,),
    in_specs=[
        pl.BlockSpec((NC, BR, 1), lambda i: (0, i, 0)),
        pl.BlockSpec((NC, BR, D), lambda i: (0, i, 0)),
        pl.BlockSpec((1, D), lambda i: (0, 0)),
    ],
    out_specs=pl.BlockSpec((BR, D), lambda i: (i, 0)),
    out_shape=jax.ShapeDtypeStruct((N, D), jnp.float32),
)


def _pack_edges(src, dst):
    npad = EPT - EPW
    k = jnp.arange(npad, dtype=jnp.int32)[None, :]
    w = jnp.arange(NW, dtype=jnp.int32)[:, None]
    pad_src = (k * 37 + w * 613) % N
    pad_dst = N + (k + w) % NSINK
    src_full = jnp.concatenate(
        [src.reshape(NW, EPW), jnp.broadcast_to(pad_src, (NW, npad))], axis=1)
    dst_full = jnp.concatenate(
        [dst.reshape(NW, EPW), jnp.broadcast_to(pad_dst, (NW, npad))], axis=1)
    return jnp.stack([src_full.reshape(NW, NCH, CH),
                      dst_full.reshape(NW, NCH, CH)], axis=2)


def kernel(x, edge_index, W, b):
    ei = edge_index.astype(jnp.int32)
    sd = _pack_edges(ei[0], ei[1])
    zeros1 = jnp.zeros((N + NSINK,), jnp.float32)
    zeros2 = jnp.zeros((N, D), jnp.float32)

    degp = _deg_kernel(sd, zeros1)
    xw = _xw_call(x, W)
    degp3 = degp[:, :N].reshape(NC, N, 1)
    y = _y_call(degp3, xw)
    accp = _agg_kernel(sd, y, zeros2)
    return _out_call(degp3, accp, b.reshape(1, D))


# speedup vs baseline: 39.2737x; 1.0002x over previous
import functools

import jax
import jax.numpy as jnp
from jax import lax
from jax.experimental import pallas as pl
from jax.experimental.pallas import tpu as pltpu
from jax.experimental.pallas import tpu_sc as plsc

N = 10000
E = 320000
D = 128

NC = 2
NS = 16
NW = NC * NS
EPW = E // NW
CH = 128
NCH = 81
EPT = NCH * CH
RING = 3
NSINK = 8

RPT = 632
RPT_LAST = N - (NS - 1) * RPT

_mesh = plsc.VectorSubcoreMesh(core_axis_name="c", subcore_axis_name="s")


@functools.partial(
    pl.kernel,
    out_type=jax.ShapeDtypeStruct((NC, N + NSINK), jnp.float32),
    mesh=_mesh,
    scratch_types=[
        pltpu.VMEM((NCH, 2, CH), jnp.int32),
        pltpu.VMEM((CH,), jnp.float32),
        pltpu.VMEM_SHARED((N + NSINK,), jnp.float32),
        pltpu.SemaphoreType.DMA,
    ],
)
def _deg_kernel(sd_hbm, zeros1_hbm, out_hbm, idx_v, ones_v, deg_sh, sem):
    cid = lax.axis_index("c")
    sid = lax.axis_index("s")
    wid = sid * NC + cid

    @pl.when(sid == 0)
    def _():
        pltpu.sync_copy(zeros1_hbm, deg_sh)

    pltpu.sync_copy(sd_hbm.at[wid], idx_v)
    for i in range(CH // 16):
        ones_v[pl.ds(i * 16, 16)] = jnp.ones((16,), jnp.float32)
    plsc.subcore_barrier()

    def fire(j, _):
        pltpu.async_copy(ones_v, deg_sh.at[idx_v.at[j, 1]], sem, add=True)
        return 0

    lax.fori_loop(0, NCH, fire, 0)

    def drain(j, _):
        pltpu.make_async_copy(ones_v, deg_sh.at[idx_v.at[j, 1]], sem).wait()
        return 0

    lax.fori_loop(0, NCH, drain, 0)

    plsc.subcore_barrier()

    @pl.when(sid == 0)
    def _():
        pltpu.sync_copy(deg_sh, out_hbm.at[cid])


@functools.partial(
    pl.kernel,
    out_type=jax.ShapeDtypeStruct((NC, N, D), jnp.float32),
    mesh=_mesh,
    scratch_types=[
        pltpu.VMEM((2, CH), jnp.int32),
        pltpu.VMEM((2, CH), jnp.int32),
        pltpu.VMEM((2, CH), jnp.int32),
        pltpu.VMEM((CH, D), jnp.float32),
        pltpu.VMEM((CH, D), jnp.float32),
        pltpu.VMEM((CH, D), jnp.float32),
        pltpu.VMEM_SHARED((N + NSINK, D), jnp.float32),
        pltpu.SemaphoreType.DMA,
        pltpu.SemaphoreType.DMA,
        pltpu.SemaphoreType.DMA,
        pltpu.SemaphoreType.DMA,
        pltpu.SemaphoreType.DMA,
        pltpu.SemaphoreType.DMA,
    ],
)
def _agg_kernel(sd_hbm, y_hbm, zeros2_hbm, out_hbm,
                idx0_v, idx1_v, idx2_v, rows0_v, rows1_v, rows2_v, acc_sh,
                semg0, semg1, semg2, semi0, semi1, semi2):
    cid = lax.axis_index("c")
    sid = lax.axis_index("s")
    wid = sid * NC + cid
    r0 = sid * RPT
    idx = [idx0_v, idx1_v, idx2_v]
    rows = [rows0_v, rows1_v, rows2_v]
    semg = [semg0, semg1, semg2]
    semi = [semi0, semi1, semi2]

    init_hbm = [y_hbm, zeros2_hbm]
    for c in range(NC):
        @pl.when((cid == c) & (sid < NS - 1))
        def _(c=c):
            pltpu.sync_copy(init_hbm[c].at[pl.ds(r0, RPT)],
                            acc_sh.at[pl.ds(r0, RPT)])

        @pl.when((cid == c) & (sid == NS - 1))
        def _(c=c):
            pltpu.sync_copy(init_hbm[c].at[pl.ds(r0, RPT_LAST)],
                            acc_sh.at[pl.ds(r0, RPT_LAST)])

    @pl.when(sid == 0)
    def _():
        pltpu.sync_copy(zeros2_hbm.at[pl.ds(0, NSINK)],
                        acc_sh.at[pl.ds(N, NSINK)])

    for b in range(RING):
        pltpu.async_copy(sd_hbm.at[wid, b], idx[b], semi[b])
    for b in range(2):
        pltpu.make_async_copy(sd_hbm.at[wid, b], idx[b], semi[b]).wait()
        pltpu.async_copy(y_hbm.at[idx[b].at[0]], rows[b], semg[b])

    plsc.subcore_barrier()

    def outer(g, _):
        for b in range(RING):
            j = g * RING + b
            nb = (b + 2) % RING

            @pl.when(j + 2 < NCH)
            def _(j=j, nb=nb):
                pltpu.make_async_copy(sd_hbm.at[wid, j + 2], idx[nb],
                                      semi[nb]).wait()
                pltpu.async_copy(y_hbm.at[idx[nb].at[0]], rows[nb], semg[nb])

            pltpu.make_async_copy(y_hbm.at[idx[b].at[0]], rows[b],
                                  semg[b]).wait()
            pltpu.sync_copy(rows[b], acc_sh.at[idx[b].at[1]], add=True)

            @pl.when(j + 3 < NCH)
            def _(j=j, b=b):
                pltpu.async_copy(sd_hbm.at[wid, j + 3], idx[b], semi[b])
        return 0

    lax.fori_loop(0, NCH // RING, outer, 0)

    plsc.subcore_barrier()

    @pl.when(sid < NS - 1)
    def _():
        pltpu.sync_copy(acc_sh.at[pl.ds(r0, RPT)],
                        out_hbm.at[cid, pl.ds(r0, RPT)])

    @pl.when(sid == NS - 1)
    def _():
        pltpu.sync_copy(acc_sh.at[pl.ds(r0, RPT_LAST)],
                        out_hbm.at[cid, pl.ds(r0, RPT_LAST)])


BR = 1000


def _y_body(deg_ref, x_ref, w_ref, y_ref):
    d = deg_ref[0, :, 0] + deg_ref[1, :, 0] + 1.0
    dinv = lax.rsqrt(d)
    y_ref[...] = jnp.dot(x_ref[...] * dinv[:, None], w_ref[...],
                         preferred_element_type=jnp.float32)


_y_call = pl.pallas_call(
    _y_body,
    grid=(N // BR,),
    in_specs=[
        pl.BlockSpec((NC, BR, 1), lambda i: (0, i, 0)),
        pl.BlockSpec((BR, D), lambda i: (i, 0)),
        pl.BlockSpec((D, D), lambda i: (0, 0)),
    ],
    out_specs=pl.BlockSpec((BR, D), lambda i: (i, 0)),
    out_shape=jax.ShapeDtypeStruct((N, D), jnp.float32),
)


def _out_body(deg_ref, acc_ref, b_ref, o_ref):
    d = deg_ref[0, :, 0] + deg_ref[1, :, 0] + 1.0
    dinv = lax.rsqrt(d)
    o_ref[...] = (acc_ref[0] + acc_ref[1]) * dinv[:, None] + b_ref[...]


_out_call = pl.pallas_call(
    _out_body,
    grid=(N // BR,),
    in_specs=[
        pl.BlockSpec((NC, BR, 1), lambda i: (0, i, 0)),
        pl.BlockSpec((NC, BR, D), lambda i: (0, i, 0)),
        pl.BlockSpec((1, D), lambda i: (0, 0)),
    ],
    out_specs=pl.BlockSpec((BR, D), lambda i: (i, 0)),
    out_shape=jax.ShapeDtypeStruct((N, D), jnp.float32),
)


def _pack_edges(src, dst):
    npad = EPT - EPW
    k = jnp.arange(npad, dtype=jnp.int32)[None, :]
    w = jnp.arange(NW, dtype=jnp.int32)[:, None]
    pad_src = (k * 37 + w * 613) % N
    pad_dst = N + (k + w) % NSINK
    src_full = jnp.concatenate(
        [src.reshape(NW, EPW), jnp.broadcast_to(pad_src, (NW, npad))], axis=1)
    dst_full = jnp.concatenate(
        [dst.reshape(NW, EPW), jnp.broadcast_to(pad_dst, (NW, npad))], axis=1)
    return jnp.stack([src_full.reshape(NW, NCH, CH),
                      dst_full.reshape(NW, NCH, CH)], axis=2)


def kernel(x, edge_index, W, b):
    ei = edge_index.astype(jnp.int32)
    sd = _pack_edges(ei[0], ei[1])
    zeros1 = jnp.zeros((N + NSINK,), jnp.float32)
    zeros2 = jnp.zeros((N, D), jnp.float32)

    degp = _deg_kernel(sd, zeros1)
    degp3 = degp[:, :N].reshape(NC, N, 1)
    y = _y_call(degp3, x, W)
    accp = _agg_kernel(sd, y, zeros2)
    return _out_call(degp3, accp, b.reshape(1, D))


# speedup vs baseline: 41.8852x; 1.0665x over previous
import functools

import jax
import jax.numpy as jnp
from jax import lax
from jax.experimental import pallas as pl
from jax.experimental.pallas import tpu as pltpu
from jax.experimental.pallas import tpu_sc as plsc

N = 10000
E = 320000
D = 128

NC = 2
NS = 16
NW = NC * NS
EPW = E // NW
CH = 128
NCH = EPW // CH
TAIL = EPW - NCH * CH
RING = 3
NSINK = 8
NCH1 = NCH + 1

RPT = 632
RPT_LAST = N - (NS - 1) * RPT

_mesh = plsc.VectorSubcoreMesh(core_axis_name="c", subcore_axis_name="s")


@functools.partial(
    pl.kernel,
    out_type=jax.ShapeDtypeStruct((NC, N + NSINK), jnp.float32),
    mesh=_mesh,
    scratch_types=[
        pltpu.VMEM((NCH1, CH), jnp.int32),
        pltpu.VMEM((CH,), jnp.float32),
        pltpu.VMEM_SHARED((N + NSINK,), jnp.float32),
        pltpu.SemaphoreType.DMA,
    ],
)
def _deg_kernel(dstp_hbm, zeros1_hbm, out_hbm, idx_v, ones_v, deg_sh, sem):
    cid = lax.axis_index("c")
    sid = lax.axis_index("s")
    wid = sid * NC + cid

    @pl.when(sid == 0)
    def _():
        pltpu.sync_copy(zeros1_hbm, deg_sh)

    pltpu.sync_copy(dstp_hbm.at[wid], idx_v)
    for i in range(CH // 16):
        ones_v[pl.ds(i * 16, 16)] = jnp.ones((16,), jnp.float32)
    plsc.subcore_barrier()

    def fire(j, _):
        pltpu.async_copy(ones_v, deg_sh.at[idx_v.at[j]], sem, add=True)
        return 0

    lax.fori_loop(0, NCH1, fire, 0)

    def drain(j, _):
        pltpu.make_async_copy(ones_v, deg_sh.at[idx_v.at[j]], sem).wait()
        return 0

    lax.fori_loop(0, NCH1, drain, 0)

    plsc.subcore_barrier()

    @pl.when(sid == 0)
    def _():
        pltpu.sync_copy(deg_sh, out_hbm.at[cid])


@functools.partial(
    pl.kernel,
    out_type=jax.ShapeDtypeStruct((NC, N, D), jnp.float32),
    mesh=_mesh,
    scratch_types=[
        pltpu.VMEM((CH,), jnp.int32),
        pltpu.VMEM((CH,), jnp.int32),
        pltpu.VMEM((CH,), jnp.int32),
        pltpu.VMEM((CH,), jnp.int32),
        pltpu.VMEM((CH,), jnp.int32),
        pltpu.VMEM((CH,), jnp.int32),
        pltpu.VMEM((TAIL,), jnp.int32),
        pltpu.VMEM((TAIL,), jnp.int32),
        pltpu.VMEM((CH, D), jnp.float32),
        pltpu.VMEM((CH, D), jnp.float32),
        pltpu.VMEM((CH, D), jnp.float32),
        pltpu.VMEM_SHARED((N, D), jnp.float32),
        pltpu.SemaphoreType.DMA,
        pltpu.SemaphoreType.DMA,
        pltpu.SemaphoreType.DMA,
        pltpu.SemaphoreType.DMA,
        pltpu.SemaphoreType.DMA,
        pltpu.SemaphoreType.DMA,
        pltpu.SemaphoreType.DMA,
    ],
)
def _agg_kernel(flat_hbm, y_hbm, zeros2_hbm, out_hbm,
                sb0, sb1, sb2, db0, db1, db2, stail_v, dtail_v,
                rows0_v, rows1_v, rows2_v, acc_sh,
                semg0, semg1, semg2, semi0, semi1, semi2, semt):
    cid = lax.axis_index("c")
    sid = lax.axis_index("s")
    wid = sid * NC + cid
    r0 = sid * RPT
    e0 = wid * EPW
    sb = [sb0, sb1, sb2]
    db = [db0, db1, db2]
    rows = [rows0_v, rows1_v, rows2_v]
    semg = [semg0, semg1, semg2]
    semi = [semi0, semi1, semi2]

    def load_idx(j, b):
        pltpu.async_copy(flat_hbm.at[pl.ds(e0 + j * CH, CH)], sb[b], semi[b])
        pltpu.async_copy(flat_hbm.at[pl.ds(E + e0 + j * CH, CH)], db[b],
                         semi[b])

    def wait_idx(j, b):
        pltpu.make_async_copy(flat_hbm.at[pl.ds(e0 + j * CH, CH)], sb[b],
                              semi[b]).wait()
        pltpu.make_async_copy(flat_hbm.at[pl.ds(E + e0 + j * CH, CH)], db[b],
                              semi[b]).wait()

    @pl.when(sid < NS - 1)
    def _():
        pltpu.sync_copy(zeros2_hbm.at[pl.ds(r0, RPT)],
                        acc_sh.at[pl.ds(r0, RPT)])

    @pl.when(sid == NS - 1)
    def _():
        pltpu.sync_copy(zeros2_hbm.at[pl.ds(r0, RPT_LAST)],
                        acc_sh.at[pl.ds(r0, RPT_LAST)])

    for b in range(RING):
        load_idx(b, b)
    pltpu.sync_copy(flat_hbm.at[pl.ds(e0 + NCH * CH, TAIL)], stail_v)
    pltpu.sync_copy(flat_hbm.at[pl.ds(E + e0 + NCH * CH, TAIL)], dtail_v)

    plsc.subcore_barrier()

    pltpu.async_copy(y_hbm.at[stail_v], rows0_v.at[pl.ds(0, TAIL)],
                     semt).wait()
    pltpu.sync_copy(rows0_v.at[pl.ds(0, TAIL)], acc_sh.at[dtail_v], add=True)

    for b in range(2):
        wait_idx(b, b)
        pltpu.async_copy(y_hbm.at[sb[b]], rows[b], semg[b])

    def outer(g, _):
        for b in range(RING):
            j = g * RING + b
            nb = (b + 2) % RING

            @pl.when(j + 2 < NCH)
            def _(j=j, nb=nb):
                wait_idx(j + 2, nb)
                pltpu.async_copy(y_hbm.at[sb[nb]], rows[nb], semg[nb])

            pltpu.make_async_copy(y_hbm.at[sb[b]], rows[b], semg[b]).wait()
            pltpu.sync_copy(rows[b], acc_sh.at[db[b]], add=True)

            @pl.when(j + 3 < NCH)
            def _(j=j, b=b):
                load_idx(j + 3, b)
        return 0

    lax.fori_loop(0, NCH // RING, outer, 0)

    plsc.subcore_barrier()

    @pl.when(sid < NS - 1)
    def _():
        pltpu.sync_copy(acc_sh.at[pl.ds(r0, RPT)],
                        out_hbm.at[cid, pl.ds(r0, RPT)])

    @pl.when(sid == NS - 1)
    def _():
        pltpu.sync_copy(acc_sh.at[pl.ds(r0, RPT_LAST)],
                        out_hbm.at[cid, pl.ds(r0, RPT_LAST)])


BR = 2000


def _y_body(deg_ref, x_ref, w_ref, y_ref):
    d = deg_ref[0, :, 0] + deg_ref[1, :, 0] + 1.0
    dinv = lax.rsqrt(d)
    y_ref[...] = jnp.dot(x_ref[...] * dinv[:, None], w_ref[...],
                         preferred_element_type=jnp.float32)


_y_call = pl.pallas_call(
    _y_body,
    grid=(N // BR,),
    in_specs=[
        pl.BlockSpec((NC, BR, 1), lambda i: (0, i, 0)),
        pl.BlockSpec((BR, D), lambda i: (i, 0)),
        pl.BlockSpec((D, D), lambda i: (0, 0)),
    ],
    out_specs=pl.BlockSpec((BR, D), lambda i: (i, 0)),
    out_shape=jax.ShapeDtypeStruct((N, D), jnp.float32),
)


def _out_body(deg_ref, acc_ref, y_ref, b_ref, o_ref):
    d = deg_ref[0, :, 0] + deg_ref[1, :, 0] + 1.0
    dinv = lax.rsqrt(d)
    o_ref[...] = ((acc_ref[0] + acc_ref[1] + y_ref[...]) * dinv[:, None]
                  + b_ref[...])


_out_call = pl.pallas_call(
    _out_body,
    grid=(N // BR,),
    in_specs=[
        pl.BlockSpec((NC, BR, 1), lambda i: (0, i, 0)),
        pl.BlockSpec((NC, BR, D), lambda i: (0, i, 0)),
        pl.BlockSpec((BR, D), lambda i: (i, 0)),
        pl.BlockSpec((1, D), lambda i: (0, 0)),
    ],
    out_specs=pl.BlockSpec((BR, D), lambda i: (i, 0)),
    out_shape=jax.ShapeDtypeStruct((N, D), jnp.float32),
)


def kernel(x, edge_index, W, b):
    ei = edge_index.astype(jnp.int32)
    flat = ei.reshape(2 * E)
    w_idx = jnp.arange(NW, dtype=jnp.int32)[:, None]
    k_idx = jnp.arange(NCH1 * CH - EPW, dtype=jnp.int32)[None, :]
    sink = N + (w_idx + k_idx) % NSINK
    dstp = jnp.concatenate(
        [ei[1].reshape(NW, EPW),
         jnp.broadcast_to(sink, (NW, NCH1 * CH - EPW))],
        axis=1).reshape(NW, NCH1, CH)
    zeros1 = jnp.zeros((N + NSINK,), jnp.float32)
    zeros2 = jnp.zeros((N, D), jnp.float32)

    degp = _deg_kernel(dstp, zeros1)
    degp3 = degp[:, :N].reshape(NC, N, 1)
    y = _y_call(degp3, x, W)
    accp = _agg_kernel(flat, y, zeros2)
    return _out_call(degp3, accp, y, b.reshape(1, D))


# speedup vs baseline: 43.2044x; 1.0315x over previous
import functools

import jax
import jax.numpy as jnp
from jax import lax
from jax.experimental import pallas as pl
from jax.experimental.pallas import tpu as pltpu
from jax.experimental.pallas import tpu_sc as plsc

N = 10000
E = 320000
D = 128

NC = 2
NS = 16
NW = NC * NS
EPW = E // NW
CH = 128
NCH = EPW // CH
TAIL = EPW - NCH * CH
RING = 3
NSINK = 8
NCH1 = NCH + 1

RPT = 632
RPT_LAST = N - (NS - 1) * RPT

_mesh = plsc.VectorSubcoreMesh(core_axis_name="c", subcore_axis_name="s")


@functools.partial(
    pl.kernel,
    out_type=jax.ShapeDtypeStruct((NC, N + NSINK), jnp.float32),
    mesh=_mesh,
    scratch_types=[
        pltpu.VMEM((NCH1, CH), jnp.int32),
        pltpu.VMEM((CH,), jnp.float32),
        pltpu.VMEM_SHARED((N + NSINK,), jnp.float32),
        pltpu.SemaphoreType.DMA,
    ],
)
def _deg_kernel(dstp_hbm, zeros1_hbm, out_hbm, idx_v, ones_v, deg_sh, sem):
    cid = lax.axis_index("c")
    sid = lax.axis_index("s")
    wid = sid * NC + cid

    @pl.when(sid == 0)
    def _():
        pltpu.sync_copy(zeros1_hbm, deg_sh)

    pltpu.sync_copy(dstp_hbm.at[wid], idx_v)
    for i in range(CH // 16):
        ones_v[pl.ds(i * 16, 16)] = jnp.ones((16,), jnp.float32)
    plsc.subcore_barrier()

    def fire(j, _):
        pltpu.async_copy(ones_v, deg_sh.at[idx_v.at[j]], sem, add=True)
        return 0

    lax.fori_loop(0, NCH1, fire, 0)

    def drain(j, _):
        pltpu.make_async_copy(ones_v, deg_sh.at[idx_v.at[j]], sem).wait()
        return 0

    lax.fori_loop(0, NCH1, drain, 0)

    plsc.subcore_barrier()

    @pl.when(sid == 0)
    def _():
        pltpu.sync_copy(deg_sh, out_hbm.at[cid])


@functools.partial(
    pl.kernel,
    out_type=jax.ShapeDtypeStruct((NC, N, D), jnp.float32),
    mesh=_mesh,
    scratch_types=[
        pltpu.VMEM((CH,), jnp.int32),
        pltpu.VMEM((CH,), jnp.int32),
        pltpu.VMEM((CH,), jnp.int32),
        pltpu.VMEM((CH,), jnp.int32),
        pltpu.VMEM((CH,), jnp.int32),
        pltpu.VMEM((CH,), jnp.int32),
        pltpu.VMEM((TAIL,), jnp.int32),
        pltpu.VMEM((TAIL,), jnp.int32),
        pltpu.VMEM((CH, D), jnp.float32),
        pltpu.VMEM((CH, D), jnp.float32),
        pltpu.VMEM((CH, D), jnp.float32),
        pltpu.VMEM_SHARED((N, D), jnp.float32),
        pltpu.SemaphoreType.DMA,
        pltpu.SemaphoreType.DMA,
        pltpu.SemaphoreType.DMA,
        pltpu.SemaphoreType.DMA,
        pltpu.SemaphoreType.DMA,
        pltpu.SemaphoreType.DMA,
        pltpu.SemaphoreType.DMA,
    ],
)
def _agg_kernel(src_hbm, dst_hbm, y_hbm, out_hbm,
                sb0, sb1, sb2, db0, db1, db2, stail_v, dtail_v,
                rows0_v, rows1_v, rows2_v, acc_sh,
                semg0, semg1, semg2, semi0, semi1, semi2, semt):
    cid = lax.axis_index("c")
    sid = lax.axis_index("s")
    wid = sid * NC + cid
    r0 = sid * RPT
    e0 = wid * EPW
    sb = [sb0, sb1, sb2]
    db = [db0, db1, db2]
    rows = [rows0_v, rows1_v, rows2_v]
    semg = [semg0, semg1, semg2]
    semi = [semi0, semi1, semi2]

    def load_idx(j, b):
        pltpu.async_copy(src_hbm.at[pl.ds(e0 + j * CH, CH)], sb[b], semi[b])
        pltpu.async_copy(dst_hbm.at[pl.ds(e0 + j * CH, CH)], db[b], semi[b])

    def wait_idx(j, b):
        pltpu.make_async_copy(src_hbm.at[pl.ds(e0 + j * CH, CH)], sb[b],
                              semi[b]).wait()
        pltpu.make_async_copy(dst_hbm.at[pl.ds(e0 + j * CH, CH)], db[b],
                              semi[b]).wait()

    for b in range(RING):
        load_idx(b, b)
    pltpu.sync_copy(src_hbm.at[pl.ds(e0 + NCH * CH, TAIL)], stail_v)
    pltpu.sync_copy(dst_hbm.at[pl.ds(e0 + NCH * CH, TAIL)], dtail_v)

    def zrow(r, _):
        for c in range(D // 16):
            rows0_v[r, pl.ds(c * 16, 16)] = jnp.zeros((16,), jnp.float32)
        return 0

    lax.fori_loop(0, CH, zrow, 0)
    for k in range(4):
        pltpu.sync_copy(rows0_v, acc_sh.at[pl.ds(r0 + k * CH, CH)])

    @pl.when(sid < NS - 1)
    def _():
        pltpu.sync_copy(rows0_v.at[pl.ds(0, RPT - 4 * CH)],
                        acc_sh.at[pl.ds(r0 + 4 * CH, RPT - 4 * CH)])

    @pl.when(sid == NS - 1)
    def _():
        pltpu.sync_copy(rows0_v.at[pl.ds(0, RPT_LAST - 4 * CH)],
                        acc_sh.at[pl.ds(r0 + 4 * CH, RPT_LAST - 4 * CH)])

    plsc.subcore_barrier()

    pltpu.async_copy(y_hbm.at[stail_v], rows0_v.at[pl.ds(0, TAIL)],
                     semt).wait()
    pltpu.sync_copy(rows0_v.at[pl.ds(0, TAIL)], acc_sh.at[dtail_v], add=True)

    for b in range(2):
        wait_idx(b, b)
        pltpu.async_copy(y_hbm.at[sb[b]], rows[b], semg[b])

    def outer(g, _):
        for b in range(RING):
            j = g * RING + b
            nb = (b + 2) % RING

            @pl.when(j + 2 < NCH)
            def _(j=j, nb=nb):
                wait_idx(j + 2, nb)
                pltpu.async_copy(y_hbm.at[sb[nb]], rows[nb], semg[nb])

            pltpu.make_async_copy(y_hbm.at[sb[b]], rows[b], semg[b]).wait()
            pltpu.sync_copy(rows[b], acc_sh.at[db[b]], add=True)

            @pl.when(j + 3 < NCH)
            def _(j=j, b=b):
                load_idx(j + 3, b)
        return 0

    lax.fori_loop(0, NCH // RING, outer, 0)

    plsc.subcore_barrier()

    @pl.when(sid < NS - 1)
    def _():
        pltpu.sync_copy(acc_sh.at[pl.ds(r0, RPT)],
                        out_hbm.at[cid, pl.ds(r0, RPT)])

    @pl.when(sid == NS - 1)
    def _():
        pltpu.sync_copy(acc_sh.at[pl.ds(r0, RPT_LAST)],
                        out_hbm.at[cid, pl.ds(r0, RPT_LAST)])


BR = 2000


def _y_body(deg_ref, x_ref, w_ref, y_ref):
    d = deg_ref[0, :, 0] + deg_ref[1, :, 0] + 1.0
    dinv = lax.rsqrt(d)
    y_ref[...] = jnp.dot(x_ref[...] * dinv[:, None], w_ref[...],
                         preferred_element_type=jnp.float32)


_y_call = pl.pallas_call(
    _y_body,
    grid=(N // BR,),
    in_specs=[
        pl.BlockSpec((NC, BR, 1), lambda i: (0, i, 0)),
        pl.BlockSpec((BR, D), lambda i: (i, 0)),
        pl.BlockSpec((D, D), lambda i: (0, 0)),
    ],
    out_specs=pl.BlockSpec((BR, D), lambda i: (i, 0)),
    out_shape=jax.ShapeDtypeStruct((N, D), jnp.float32),
)


def _out_body(deg_ref, acc_ref, y_ref, b_ref, o_ref):
    d = deg_ref[0, :, 0] + deg_ref[1, :, 0] + 1.0
    dinv = lax.rsqrt(d)
    o_ref[...] = ((acc_ref[0] + acc_ref[1] + y_ref[...]) * dinv[:, None]
                  + b_ref[...])


_out_call = pl.pallas_call(
    _out_body,
    grid=(N // BR,),
    in_specs=[
        pl.BlockSpec((NC, BR, 1), lambda i: (0, i, 0)),
        pl.BlockSpec((NC, BR, D), lambda i: (0, i, 0)),
        pl.BlockSpec((BR, D), lambda i: (i, 0)),
        pl.BlockSpec((1, D), lambda i: (0, 0)),
    ],
    out_specs=pl.BlockSpec((BR, D), lambda i: (i, 0)),
    out_shape=jax.ShapeDtypeStruct((N, D), jnp.float32),
)


def kernel(x, edge_index, W, b):
    ei = edge_index.astype(jnp.int32)
    src1d = ei[0]
    dst1d = ei[1]
    w_idx = jnp.arange(NW, dtype=jnp.int32)[:, None]
    k_idx = jnp.arange(NCH1 * CH - EPW, dtype=jnp.int32)[None, :]
    sink = N + (w_idx + k_idx) % NSINK
    dstp = jnp.concatenate(
        [dst1d.reshape(NW, EPW),
         jnp.broadcast_to(sink, (NW, NCH1 * CH - EPW))],
        axis=1).reshape(NW, NCH1, CH)
    zeros1 = jnp.zeros((N + NSINK,), jnp.float32)

    degp = _deg_kernel(dstp, zeros1)
    degp3 = degp[:, :N].reshape(NC, N, 1)
    y = _y_call(degp3, x, W)
    accp = _agg_kernel(src1d, dst1d, y)
    return _out_call(degp3, accp, y, b.reshape(1, D))


# speedup vs baseline: 46.8749x; 1.0850x over previous
import functools

import jax
import jax.numpy as jnp
from jax import lax
from jax.experimental import pallas as pl
from jax.experimental.pallas import tpu as pltpu
from jax.experimental.pallas import tpu_sc as plsc

N = 10000
E = 320000
D = 128

NC = 2
NS = 16
NW = NC * NS
CH = 128
NCHG = E // CH
NCH = NCHG // NW
NEXTRA = NCHG - NCH * NW
RING = 3

RPT = 632
RPT_LAST = N - (NS - 1) * RPT

_mesh = plsc.VectorSubcoreMesh(core_axis_name="c", subcore_axis_name="s")


@functools.partial(
    pl.kernel,
    out_type=jax.ShapeDtypeStruct((NC, N), jnp.float32),
    mesh=_mesh,
    scratch_types=[
        pltpu.VMEM((NCH + 1, CH), jnp.int32),
        pltpu.VMEM((CH,), jnp.float32),
        pltpu.VMEM_SHARED((N,), jnp.float32),
        pltpu.SemaphoreType.DMA,
        pltpu.SemaphoreType.DMA,
    ],
)
def _deg_kernel(ei_hbm, zeros1_hbm, out_hbm, idx_v, ones_v, deg_sh,
                seml, sems):
    cid = lax.axis_index("c")
    sid = lax.axis_index("s")
    wid = sid * NC + cid
    c0 = wid * NCH

    @pl.when(sid == 0)
    def _():
        pltpu.sync_copy(zeros1_hbm, deg_sh)

    def fire_load(j, _):
        pltpu.async_copy(ei_hbm.at[1, pl.ds((c0 + j) * CH, CH)],
                         idx_v.at[j], seml)
        return 0

    lax.fori_loop(0, NCH, fire_load, 0)

    @pl.when(wid < NEXTRA)
    def _():
        pltpu.async_copy(ei_hbm.at[1, pl.ds((NCH * NW + wid) * CH, CH)],
                         idx_v.at[NCH], seml)

    for i in range(CH // 16):
        ones_v[pl.ds(i * 16, 16)] = jnp.ones((16,), jnp.float32)

    def drain_load(j, _):
        pltpu.make_async_copy(ei_hbm.at[1, pl.ds((c0 + j) * CH, CH)],
                              idx_v.at[j], seml).wait()
        return 0

    lax.fori_loop(0, NCH, drain_load, 0)

    @pl.when(wid < NEXTRA)
    def _():
        pltpu.make_async_copy(ei_hbm.at[1, pl.ds((NCH * NW + wid) * CH, CH)],
                              idx_v.at[NCH], seml).wait()

    plsc.subcore_barrier()

    def fire_scatter(j, _):
        pltpu.async_copy(ones_v, deg_sh.at[idx_v.at[j]], sems, add=True)
        return 0

    lax.fori_loop(0, NCH, fire_scatter, 0)

    @pl.when(wid < NEXTRA)
    def _():
        pltpu.async_copy(ones_v, deg_sh.at[idx_v.at[NCH]], sems, add=True)

    def drain_scatter(j, _):
        pltpu.make_async_copy(ones_v, deg_sh.at[idx_v.at[j]], sems).wait()
        return 0

    lax.fori_loop(0, NCH, drain_scatter, 0)

    @pl.when(wid < NEXTRA)
    def _():
        pltpu.make_async_copy(ones_v, deg_sh.at[idx_v.at[NCH]], sems).wait()

    plsc.subcore_barrier()

    @pl.when(sid == 0)
    def _():
        pltpu.sync_copy(deg_sh, out_hbm.at[cid])


@functools.partial(
    pl.kernel,
    out_type=jax.ShapeDtypeStruct((NC, N, D), jnp.float32),
    mesh=_mesh,
    scratch_types=[
        pltpu.VMEM((CH,), jnp.int32),
        pltpu.VMEM((CH,), jnp.int32),
        pltpu.VMEM((CH,), jnp.int32),
        pltpu.VMEM((CH,), jnp.int32),
        pltpu.VMEM((CH,), jnp.int32),
        pltpu.VMEM((CH,), jnp.int32),
        pltpu.VMEM((CH, D), jnp.float32),
        pltpu.VMEM((CH, D), jnp.float32),
        pltpu.VMEM((CH, D), jnp.float32),
        pltpu.VMEM_SHARED((N, D), jnp.float32),
        pltpu.SemaphoreType.DMA,
        pltpu.SemaphoreType.DMA,
        pltpu.SemaphoreType.DMA,
        pltpu.SemaphoreType.DMA,
        pltpu.SemaphoreType.DMA,
        pltpu.SemaphoreType.DMA,
    ],
)
def _agg_kernel(ei_hbm, y_hbm, out_hbm,
                sb0, sb1, sb2, db0, db1, db2,
                rows0_v, rows1_v, rows2_v, acc_sh,
                semg0, semg1, semg2, semi0, semi1, semi2):
    cid = lax.axis_index("c")
    sid = lax.axis_index("s")
    wid = sid * NC + cid
    r0 = sid * RPT
    c0 = wid * NCH
    sb = [sb0, sb1, sb2]
    db = [db0, db1, db2]
    rows = [rows0_v, rows1_v, rows2_v]
    semg = [semg0, semg1, semg2]
    semi = [semi0, semi1, semi2]

    def load_idx(c, b):
        pltpu.async_copy(ei_hbm.at[0, pl.ds(c * CH, CH)], sb[b], semi[b])
        pltpu.async_copy(ei_hbm.at[1, pl.ds(c * CH, CH)], db[b], semi[b])

    def wait_idx(c, b):
        pltpu.make_async_copy(ei_hbm.at[0, pl.ds(c * CH, CH)], sb[b],
                              semi[b]).wait()
        pltpu.make_async_copy(ei_hbm.at[1, pl.ds(c * CH, CH)], db[b],
                              semi[b]).wait()

    for b in range(RING):
        load_idx(c0 + b, b)

    def zrow(r, _):
        for c in range(D // 16):
            rows0_v[r, pl.ds(c * 16, 16)] = jnp.zeros((16,), jnp.float32)
        return 0

    lax.fori_loop(0, CH, zrow, 0)
    for k in range(4):
        pltpu.sync_copy(rows0_v, acc_sh.at[pl.ds(r0 + k * CH, CH)])

    @pl.when(sid < NS - 1)
    def _():
        pltpu.sync_copy(rows0_v.at[pl.ds(0, RPT - 4 * CH)],
                        acc_sh.at[pl.ds(r0 + 4 * CH, RPT - 4 * CH)])

    @pl.when(sid == NS - 1)
    def _():
        pltpu.sync_copy(rows0_v.at[pl.ds(0, RPT_LAST - 4 * CH)],
                        acc_sh.at[pl.ds(r0 + 4 * CH, RPT_LAST - 4 * CH)])

    plsc.subcore_barrier()

    for b in range(2):
        wait_idx(c0 + b, b)
        pltpu.async_copy(y_hbm.at[sb[b]], rows[b], semg[b])

    def outer(g, _):
        for b in range(RING):
            j = g * RING + b
            nb = (b + 2) % RING

            @pl.when(j + 2 < NCH)
            def _(j=j, nb=nb):
                wait_idx(c0 + j + 2, nb)
                pltpu.async_copy(y_hbm.at[sb[nb]], rows[nb], semg[nb])

            pltpu.make_async_copy(y_hbm.at[sb[b]], rows[b], semg[b]).wait()
            pltpu.sync_copy(rows[b], acc_sh.at[db[b]], add=True)

            @pl.when(j + 3 < NCH)
            def _(j=j, b=b):
                load_idx(c0 + j + 3, b)
        return 0

    lax.fori_loop(0, NCH // RING, outer, 0)

    @pl.when(wid < NEXTRA)
    def _():
        ce = NCH * NW + wid
        load_idx(ce, 0)
        wait_idx(ce, 0)
        pltpu.async_copy(y_hbm.at[sb[0]], rows[0], semg[0])
        pltpu.make_async_copy(y_hbm.at[sb[0]], rows[0], semg[0]).wait()
        pltpu.sync_copy(rows[0], acc_sh.at[db[0]], add=True)

    plsc.subcore_barrier()

    @pl.when(sid < NS - 1)
    def _():
        pltpu.sync_copy(acc_sh.at[pl.ds(r0, RPT)],
                        out_hbm.at[cid, pl.ds(r0, RPT)])

    @pl.when(sid == NS - 1)
    def _():
        pltpu.sync_copy(acc_sh.at[pl.ds(r0, RPT_LAST)],
                        out_hbm.at[cid, pl.ds(r0, RPT_LAST)])


BR = 2000


def _y_body(deg_ref, x_ref, w_ref, y_ref):
    d = deg_ref[0, :, 0] + deg_ref[1, :, 0] + 1.0
    dinv = lax.rsqrt(d)
    y_ref[...] = jnp.dot(x_ref[...] * dinv[:, None], w_ref[...],
                         preferred_element_type=jnp.float32)


_y_call = pl.pallas_call(
    _y_body,
    grid=(N // BR,),
    in_specs=[
        pl.BlockSpec((NC, BR, 1), lambda i: (0, i, 0)),
        pl.BlockSpec((BR, D), lambda i: (i, 0)),
        pl.BlockSpec((D, D), lambda i: (0, 0)),
    ],
    out_specs=pl.BlockSpec((BR, D), lambda i: (i, 0)),
    out_shape=jax.ShapeDtypeStruct((N, D), jnp.float32),
)


def _out_body(deg_ref, acc_ref, y_ref, b_ref, o_ref):
    d = deg_ref[0, :, 0] + deg_ref[1, :, 0] + 1.0
    dinv = lax.rsqrt(d)
    o_ref[...] = ((acc_ref[0] + acc_ref[1] + y_ref[...]) * dinv[:, None]
                  + b_ref[...])


_out_call = pl.pallas_call(
    _out_body,
    grid=(N // BR,),
    in_specs=[
        pl.BlockSpec((NC, BR, 1), lambda i: (0, i, 0)),
        pl.BlockSpec((NC, BR, D), lambda i: (0, i, 0)),
        pl.BlockSpec((BR, D), lambda i: (i, 0)),
        pl.BlockSpec((1, D), lambda i: (0, 0)),
    ],
    out_specs=pl.BlockSpec((BR, D), lambda i: (i, 0)),
    out_shape=jax.ShapeDtypeStruct((N, D), jnp.float32),
)


def kernel(x, edge_index, W, b):
    ei = edge_index.astype(jnp.int32)
    zeros1 = jnp.zeros((N,), jnp.float32)

    degp = _deg_kernel(ei, zeros1)
    degp3 = degp.reshape(NC, N, 1)
    y = _y_call(degp3, x, W)
    accp = _agg_kernel(ei, y)
    return _out_call(degp3, accp, y, b.reshape(1, D))
